# baseline (speedup 1.0000x reference)
import dataclasses
import functools

import jax
import jax.numpy as jnp
from jax import lax
from jax.experimental import pallas as pl
from jax.experimental.pallas import tpu as pltpu
from jax.experimental.pallas import tpu_sc as plsc

H, W, D = 256, 704, 112
D2 = 128
N = 100000
NC, NS, L = 2, 16, 16
PTS = 6272
NPAD = PTS * NS
CHUNK_ROWS = 8
CPC = H // (NC * CHUNK_ROWS)
NG = CHUNK_ROWS + 1
CH = NG * W + 16
ZR = 88
OSTRIPE = CHUNK_ROWS * W // NS
PSTRIPE = NG * W // NS
K = 32
NV = PTS // L


def _splat_body(xs_hbm, ys_hbm, feat_hbm, out_hbm,
                xs_t, ys_t, sel_t, fbuf0, fbuf1, fbuf2, fbuf3, sbuf0, sbuf1,
                ibuf0, ibuf1, zbuf, spmem,
                gsem0, gsem1, gsem2, gsem3, ssem0, ssem1, zsem):
    c = lax.axis_index("c")
    s = lax.axis_index("s")
    sbase = s * PTS

    pltpu.sync_copy(xs_hbm.at[pl.ds(sbase, PTS)], xs_t)
    pltpu.sync_copy(ys_hbm.at[pl.ds(sbase, PTS)], ys_t)

    zv = jnp.zeros((L,), jnp.float32)

    @pl.loop(0, ZR)
    def _(r):
        @pl.loop(0, D2 // L)
        def _(j):
            zbuf[r, pl.ds(j * L, L)] = zv

    def splat(x):
        return lax.broadcast_in_dim(x, (L,), ())

    sbase_v = lax.broadcast_in_dim(sbase, (L,), ())
    zrow = jnp.zeros((L,), jnp.int32)
    zf = jnp.zeros((L,), jnp.float32)

    def select_chunk(ci):
        lof = ((c * CPC + ci) * CHUNK_ROWS) * 1.0
        ext = jnp.where((c == 1) & (ci == 0), 1.0, 0.0)

        def sel_body(i, cnt):
            off = i * L
            yv = ys_t[pl.ds(off, L)]
            sel = (yv >= lof - ext) & (yv < lof + CHUNK_ROWS)
            seli = sel.astype(jnp.int32)
            pos = jnp.cumsum(seli)
            ids = lax.iota(jnp.int32, L) + (sbase + off)
            addr = pos + (cnt - 1)
            plsc.store_scatter(sel_t, [addr], ids, mask=sel)
            return cnt + jnp.sum(seli)

        cnt = lax.fori_loop(0, NV, sel_body, 0)

        for t in range(4 * K // L):
            addr = lax.iota(jnp.int32, L) + (cnt + t * L)
            plsc.store_scatter(sel_t, [addr], sbase_v)
        return cnt

    for z in range(4):
        pltpu.sync_copy(zbuf, spmem.at[pl.ds(s * PSTRIPE + z * ZR, ZR)])
    pltpu.sync_copy(zbuf.at[pl.ds(0, PSTRIPE - 4 * ZR)],
                    spmem.at[pl.ds(s * PSTRIPE + 4 * ZR, PSTRIPE - 4 * ZR)])

    @pl.when(s == 0)
    def _():
        pltpu.sync_copy(zbuf.at[pl.ds(0, 16)],
                        spmem.at[pl.ds(NS * PSTRIPE, 16)])

    plsc.subcore_barrier()
    cnt0 = select_chunk(0)

    def chunk_body(ci, cnt):
        lo = (c * CPC + ci) * CHUNK_ROWS
        phase = lax.rem(CHUNK_ROWS * ci, NG)

        nb = (cnt + K - 1) // K
        nq = (nb + 3) // 4
        nbv = 4 * nq

        def gather_start(b, fb, gsem):
            pltpu.async_copy(feat_hbm.at[sel_t.at[pl.ds(b * K, K)]], fb, gsem)

        def gather_wait(b, fb, gsem):
            pltpu.make_async_copy(
                feat_hbm.at[sel_t.at[pl.ds(b * K, K)]], fb, gsem).wait()

        def scatter_start(sb, ib, ssem):
            pltpu.async_copy(sb, spmem.at[ib.at[0]], ssem, add=True)

        def scatter_wait(sb, ib, ssem):
            pltpu.make_async_copy(sb, spmem.at[ib.at[0]], ssem).wait()

        def do_batch(b, fbuf, sbuf, ibuf):
            for g in range(K // L):
                idsv = sel_t[pl.ds(b * K + g * L, L)]
                lids = idsv - sbase
                xv = plsc.load_gather(xs_t, [lids])
                yv = plsc.load_gather(ys_t, [lids])
                x0v = xv.astype(jnp.int32)
                y0v = yv.astype(jnp.int32)
                wx1v = xv - x0v.astype(jnp.float32)
                wy1v = yv - y0v.astype(jnp.float32)
                slotv = lax.iota(jnp.int32, L) + (b * K + g * L)
                validv = slotv < cnt
                y0inv = validv & (y0v >= lo)
                wx0v = 1.0 - wx1v
                wy0v = 1.0 - wy1v
                c00v = jnp.where(y0inv, wx0v * wy0v, zf)
                c10v = jnp.where(y0inv, wx1v * wy0v, zf)
                c01v = jnp.where(validv, wx0v * wy1v, zf)
                c11v = jnp.where(validv, wx1v * wy1v, zf)
                t0 = y0v - lo + phase
                g0 = jnp.where(t0 >= NG, t0 - NG, t0)
                t1 = t0 + 1
                g1 = jnp.where(t1 >= NG, t1 - NG, t1)
                r0v = jnp.where(y0inv, g0 * W + x0v, zrow)
                r1v = jnp.where(validv, g1 * W + x0v, zrow)
                a0 = lax.iota(jnp.int32, L) * 4 + (g * L * 4)
                plsc.store_scatter(ibuf, [zrow, a0], r0v)
                plsc.store_scatter(ibuf, [zrow, a0 + 1], r0v + 1)
                plsc.store_scatter(ibuf, [zrow, a0 + 2], r1v)
                plsc.store_scatter(ibuf, [zrow, a0 + 3], r1v + 1)
                for k2 in range(L):
                    kk = g * L + k2
                    c00 = splat(c00v[k2])
                    c10 = splat(c10v[k2])
                    c01 = splat(c01v[k2])
                    c11 = splat(c11v[k2])
                    for j in range(D2 // L):
                        f = fbuf[kk, pl.ds(j * L, L)]
                        sbuf[4 * kk + 0, pl.ds(j * L, L)] = f * c00
                        sbuf[4 * kk + 1, pl.ds(j * L, L)] = f * c10
                        sbuf[4 * kk + 2, pl.ds(j * L, L)] = f * c01
                        sbuf[4 * kk + 3, pl.ds(j * L, L)] = f * c11

        fbufs = (fbuf0, fbuf1, fbuf2, fbuf3)
        gsems = (gsem0, gsem1, gsem2, gsem3)
        sbufs = (sbuf0, sbuf1)
        ibufs = (ibuf0, ibuf1)
        ssems = (ssem0, ssem1)

        @pl.when(nq > 0)
        def _():
            for u in range(4):
                gather_start(u, fbufs[u], gsems[u])

        def quad_body(q, carry2):
            b = 4 * q
            for u in range(4):
                gather_wait(b + u, fbufs[u], gsems[u])

                @pl.when((q > 0) | (u >= 2))
                def _():
                    scatter_wait(sbufs[u % 2], ibufs[u % 2], ssems[u % 2])

                do_batch(b + u, fbufs[u], sbufs[u % 2], ibufs[u % 2])
                scatter_start(sbufs[u % 2], ibufs[u % 2], ssems[u % 2])

                @pl.when(b + u + 4 < nbv)
                def _():
                    gather_start(b + u + 4, fbufs[u], gsems[u])

            return carry2

        lax.fori_loop(0, nq, quad_body, 0)

        @pl.when(nq > 0)
        def _():
            scatter_wait(sbuf0, ibuf0, ssem0)
            scatter_wait(sbuf1, ibuf1, ssem1)

        plsc.subcore_barrier()

        kk_ = s // 2
        hh_ = lax.rem(s, 2)
        ga_ = phase + kk_
        ga_ = jnp.where(ga_ >= NG, ga_ - NG, ga_)
        sp_off = ga_ * W + hh_ * OSTRIPE
        pltpu.sync_copy(spmem.at[pl.ds(sp_off, OSTRIPE)],
                        out_hbm.at[pl.ds((lo + kk_) * W + hh_ * OSTRIPE,
                                         OSTRIPE)])
        for z in range(OSTRIPE // ZR):
            pltpu.async_copy(zbuf, spmem.at[pl.ds(sp_off + z * ZR, ZR)],
                             zsem)
        cnt_next = select_chunk(ci + 1)
        for z in range(OSTRIPE // ZR):
            pltpu.make_async_copy(
                zbuf, spmem.at[pl.ds(sp_off + z * ZR, ZR)], zsem).wait()
        plsc.subcore_barrier()
        return cnt_next

    lax.fori_loop(0, CPC, chunk_body, cnt0)


_cp = pltpu.CompilerParams()
if "needs_layout_passes" in pltpu.CompilerParams.__dataclass_fields__:
    _cp = dataclasses.replace(_cp, needs_layout_passes=False)


@functools.partial(
    pl.kernel,
    compiler_params=_cp,
    out_type=jax.ShapeDtypeStruct((H * W, D2), jnp.float32),
    mesh=plsc.VectorSubcoreMesh(core_axis_name="c", subcore_axis_name="s"),
    scratch_types=[
        pltpu.VMEM((PTS,), jnp.float32),
        pltpu.VMEM((PTS,), jnp.float32),
        pltpu.VMEM((PTS + 5 * K,), jnp.int32),
        pltpu.VMEM((K, D2), jnp.float32),
        pltpu.VMEM((K, D2), jnp.float32),
        pltpu.VMEM((K, D2), jnp.float32),
        pltpu.VMEM((K, D2), jnp.float32),
        pltpu.VMEM((4 * K, D2), jnp.float32),
        pltpu.VMEM((4 * K, D2), jnp.float32),
        pltpu.VMEM((1, 4 * K), jnp.int32),
        pltpu.VMEM((1, 4 * K), jnp.int32),
        pltpu.VMEM((ZR, D2), jnp.float32),
        pltpu.VMEM_SHARED((CH, D2), jnp.float32),
        pltpu.SemaphoreType.DMA,
        pltpu.SemaphoreType.DMA,
        pltpu.SemaphoreType.DMA,
        pltpu.SemaphoreType.DMA,
        pltpu.SemaphoreType.DMA,
        pltpu.SemaphoreType.DMA,
        pltpu.SemaphoreType.DMA,
    ],
)
def _splat_kernel(xs_hbm, ys_hbm, feat_hbm, out_hbm, *scratch):
    _splat_body(xs_hbm, ys_hbm, feat_hbm, out_hbm, *scratch)


def _pad_tc(feats):
    blk = 2000

    def body(x_ref, o_ref):
        o_ref[...] = jnp.concatenate(
            [x_ref[...], jnp.zeros((blk, D2 - D), jnp.float32)], axis=1)

    return pl.pallas_call(
        body,
        grid=(N // blk,),
        in_specs=[pl.BlockSpec((blk, D), lambda i: (i, 0))],
        out_specs=pl.BlockSpec((blk, D2), lambda i: (i, 0)),
        out_shape=jax.ShapeDtypeStruct((N, D2), jnp.float32),
    )(feats)


def kernel(sampling_positions, sampling_depth_features, feature_shape):
    del feature_shape
    x = sampling_positions[:, 0]
    y = sampling_positions[:, 1]
    pad = NPAD - N
    xs = jnp.concatenate([x, jnp.zeros((pad,), jnp.float32)])
    ys = jnp.concatenate([y, jnp.full((pad,), 4.0e6, jnp.float32)])
    feats = _pad_tc(sampling_depth_features)
    out = _splat_kernel(xs, ys, feats)
    return out[:, :D].reshape(H, W, D)


# speedup vs baseline: 1.0290x; 1.0290x over previous
import dataclasses
import functools

import jax
import jax.numpy as jnp
from jax import lax
from jax.experimental import pallas as pl
from jax.experimental.pallas import tpu as pltpu
from jax.experimental.pallas import tpu_sc as plsc

H, W, D = 256, 704, 112
D2 = 128
N = 100000
NC, NS, L = 2, 16, 16
PTS = 6272
NPAD = PTS * NS
CHUNK_ROWS = 8
CPC = H // (NC * CHUNK_ROWS)
NG = CHUNK_ROWS + 1
CH = NG * W + 16
ZR = 88
OSTRIPE = CHUNK_ROWS * W // NS
PSTRIPE = NG * W // NS
K = 32
NV = PTS // L


def _splat_body(xs_hbm, ys_hbm, feat_hbm, out_hbm,
                xs_t, ys_t, sel_t, fbuf0, fbuf1, sbuf0, sbuf1,
                ibuf0, ibuf1, zbuf, spmem,
                gsem0, gsem1, ssem0, ssem1, zsem):
    c = lax.axis_index("c")
    s = lax.axis_index("s")
    sbase = s * PTS

    pltpu.sync_copy(xs_hbm.at[pl.ds(sbase, PTS)], xs_t)
    pltpu.sync_copy(ys_hbm.at[pl.ds(sbase, PTS)], ys_t)

    zv = jnp.zeros((L,), jnp.float32)

    @pl.loop(0, ZR)
    def _(r):
        @pl.loop(0, D2 // L)
        def _(j):
            zbuf[r, pl.ds(j * L, L)] = zv

    def splat(x):
        return lax.broadcast_in_dim(x, (L,), ())

    sbase_v = lax.broadcast_in_dim(sbase, (L,), ())
    zrow = jnp.zeros((L,), jnp.int32)
    zf = jnp.zeros((L,), jnp.float32)

    def select_chunk(ci):
        lof = ((c * CPC + ci) * CHUNK_ROWS) * 1.0
        ext = jnp.where((c == 1) & (ci == 0), 1.0, 0.0)

        def sel_body(i, cnt):
            off = i * L
            yv = ys_t[pl.ds(off, L)]
            sel = (yv >= lof - ext) & (yv < lof + CHUNK_ROWS)
            ids = lax.iota(jnp.int32, L) + (sbase + off)
            plsc.store_compressed(sel_t.at[pl.ds(cnt, L)], ids, mask=sel)
            pc = plsc.all_reduce_population_count(sel)
            return cnt + pc[0]

        cnt = lax.fori_loop(0, NV, sel_body, 0)

        for t in range(2 * K // L):
            addr = lax.iota(jnp.int32, L) + (cnt + t * L)
            plsc.store_scatter(sel_t, [addr], sbase_v)
        return cnt

    for z in range(4):
        pltpu.sync_copy(zbuf, spmem.at[pl.ds(s * PSTRIPE + z * ZR, ZR)])
    pltpu.sync_copy(zbuf.at[pl.ds(0, PSTRIPE - 4 * ZR)],
                    spmem.at[pl.ds(s * PSTRIPE + 4 * ZR, PSTRIPE - 4 * ZR)])

    @pl.when(s == 0)
    def _():
        pltpu.sync_copy(zbuf.at[pl.ds(0, 16)],
                        spmem.at[pl.ds(NS * PSTRIPE, 16)])

    plsc.subcore_barrier()
    cnt0 = select_chunk(0)

    def chunk_body(ci, cnt):
        lo = (c * CPC + ci) * CHUNK_ROWS
        phase = lax.rem(CHUNK_ROWS * ci, NG)

        nb = (cnt + K - 1) // K
        npairs = (nb + 1) // 2
        nbv = 2 * npairs

        def gather_start(b, fb, gsem):
            pltpu.async_copy(feat_hbm.at[sel_t.at[pl.ds(b * K, K)]], fb, gsem)

        def gather_wait(b, fb, gsem):
            pltpu.make_async_copy(
                feat_hbm.at[sel_t.at[pl.ds(b * K, K)]], fb, gsem).wait()

        def scatter_start(sb, ib, ssem):
            pltpu.async_copy(sb, spmem.at[ib.at[0]], ssem, add=True)

        def scatter_wait(sb, ib, ssem):
            pltpu.make_async_copy(sb, spmem.at[ib.at[0]], ssem).wait()

        def do_batch(b, fbuf, sbuf, ibuf):
            for g in range(K // L):
                idsv = sel_t[pl.ds(b * K + g * L, L)]
                lids = idsv - sbase
                xv = plsc.load_gather(xs_t, [lids])
                yv = plsc.load_gather(ys_t, [lids])
                x0v = xv.astype(jnp.int32)
                y0v = yv.astype(jnp.int32)
                wx1v = xv - x0v.astype(jnp.float32)
                wy1v = yv - y0v.astype(jnp.float32)
                slotv = lax.iota(jnp.int32, L) + (b * K + g * L)
                validv = slotv < cnt
                y0inv = validv & (y0v >= lo)
                wx0v = 1.0 - wx1v
                wy0v = 1.0 - wy1v
                c00v = jnp.where(y0inv, wx0v * wy0v, zf)
                c10v = jnp.where(y0inv, wx1v * wy0v, zf)
                c01v = jnp.where(validv, wx0v * wy1v, zf)
                c11v = jnp.where(validv, wx1v * wy1v, zf)
                t0 = y0v - lo + phase
                g0 = jnp.where(t0 >= NG, t0 - NG, t0)
                t1 = t0 + 1
                g1 = jnp.where(t1 >= NG, t1 - NG, t1)
                r0v = jnp.where(y0inv, g0 * W + x0v, zrow)
                r1v = jnp.where(validv, g1 * W + x0v, zrow)
                a0 = lax.iota(jnp.int32, L) * 4 + (g * L * 4)
                plsc.store_scatter(ibuf, [zrow, a0], r0v)
                plsc.store_scatter(ibuf, [zrow, a0 + 1], r0v + 1)
                plsc.store_scatter(ibuf, [zrow, a0 + 2], r1v)
                plsc.store_scatter(ibuf, [zrow, a0 + 3], r1v + 1)
                for k2 in range(L):
                    kk = g * L + k2
                    c00 = splat(c00v[k2])
                    c10 = splat(c10v[k2])
                    c01 = splat(c01v[k2])
                    c11 = splat(c11v[k2])
                    for j in range(D2 // L):
                        f = fbuf[kk, pl.ds(j * L, L)]
                        sbuf[4 * kk + 0, pl.ds(j * L, L)] = f * c00
                        sbuf[4 * kk + 1, pl.ds(j * L, L)] = f * c10
                        sbuf[4 * kk + 2, pl.ds(j * L, L)] = f * c01
                        sbuf[4 * kk + 3, pl.ds(j * L, L)] = f * c11

        @pl.when(npairs > 0)
        def _():
            gather_start(0, fbuf0, gsem0)
            gather_start(1, fbuf1, gsem1)

        def pair_body(p, carry2):
            b0 = 2 * p
            b1 = b0 + 1
            gather_wait(b0, fbuf0, gsem0)

            @pl.when(p > 0)
            def _():
                scatter_wait(sbuf0, ibuf0, ssem0)

            do_batch(b0, fbuf0, sbuf0, ibuf0)
            scatter_start(sbuf0, ibuf0, ssem0)

            @pl.when(b0 + 2 < nbv)
            def _():
                gather_start(b0 + 2, fbuf0, gsem0)

            gather_wait(b1, fbuf1, gsem1)

            @pl.when(p > 0)
            def _():
                scatter_wait(sbuf1, ibuf1, ssem1)

            do_batch(b1, fbuf1, sbuf1, ibuf1)
            scatter_start(sbuf1, ibuf1, ssem1)

            @pl.when(b1 + 2 < nbv)
            def _():
                gather_start(b1 + 2, fbuf1, gsem1)

            return carry2

        lax.fori_loop(0, npairs, pair_body, 0)

        @pl.when(npairs > 0)
        def _():
            scatter_wait(sbuf0, ibuf0, ssem0)
            scatter_wait(sbuf1, ibuf1, ssem1)

        plsc.subcore_barrier()

        kk_ = s // 2
        hh_ = lax.rem(s, 2)
        ga_ = phase + kk_
        ga_ = jnp.where(ga_ >= NG, ga_ - NG, ga_)
        sp_off = ga_ * W + hh_ * OSTRIPE
        pltpu.sync_copy(spmem.at[pl.ds(sp_off, OSTRIPE)],
                        out_hbm.at[pl.ds((lo + kk_) * W + hh_ * OSTRIPE,
                                         OSTRIPE)])
        for z in range(OSTRIPE // ZR):
            pltpu.async_copy(zbuf, spmem.at[pl.ds(sp_off + z * ZR, ZR)],
                             zsem)
        cnt_next = select_chunk(ci + 1)
        for z in range(OSTRIPE // ZR):
            pltpu.make_async_copy(
                zbuf, spmem.at[pl.ds(sp_off + z * ZR, ZR)], zsem).wait()
        plsc.subcore_barrier()
        return cnt_next

    lax.fori_loop(0, CPC, chunk_body, cnt0)


_cp = pltpu.CompilerParams()
if "needs_layout_passes" in pltpu.CompilerParams.__dataclass_fields__:
    _cp = dataclasses.replace(_cp, needs_layout_passes=False)


@functools.partial(
    pl.kernel,
    compiler_params=_cp,
    out_type=jax.ShapeDtypeStruct((H * W, D2), jnp.float32),
    mesh=plsc.VectorSubcoreMesh(core_axis_name="c", subcore_axis_name="s"),
    scratch_types=[
        pltpu.VMEM((PTS,), jnp.float32),
        pltpu.VMEM((PTS,), jnp.float32),
        pltpu.VMEM((PTS + 3 * K,), jnp.int32),
        pltpu.VMEM((K, D2), jnp.float32),
        pltpu.VMEM((K, D2), jnp.float32),
        pltpu.VMEM((4 * K, D2), jnp.float32),
        pltpu.VMEM((4 * K, D2), jnp.float32),
        pltpu.VMEM((1, 4 * K), jnp.int32),
        pltpu.VMEM((1, 4 * K), jnp.int32),
        pltpu.VMEM((ZR, D2), jnp.float32),
        pltpu.VMEM_SHARED((CH, D2), jnp.float32),
        pltpu.SemaphoreType.DMA,
        pltpu.SemaphoreType.DMA,
        pltpu.SemaphoreType.DMA,
        pltpu.SemaphoreType.DMA,
        pltpu.SemaphoreType.DMA,
    ],
)
def _splat_kernel(xs_hbm, ys_hbm, feat_hbm, out_hbm, *scratch):
    _splat_body(xs_hbm, ys_hbm, feat_hbm, out_hbm, *scratch)


def _pad_tc(feats):
    blk = 2000

    def body(x_ref, o_ref):
        o_ref[...] = jnp.concatenate(
            [x_ref[...], jnp.zeros((blk, D2 - D), jnp.float32)], axis=1)

    return pl.pallas_call(
        body,
        grid=(N // blk,),
        in_specs=[pl.BlockSpec((blk, D), lambda i: (i, 0))],
        out_specs=pl.BlockSpec((blk, D2), lambda i: (i, 0)),
        out_shape=jax.ShapeDtypeStruct((N, D2), jnp.float32),
    )(feats)


def kernel(sampling_positions, sampling_depth_features, feature_shape):
    del feature_shape
    x = sampling_positions[:, 0]
    y = sampling_positions[:, 1]
    pad = NPAD - N
    xs = jnp.concatenate([x, jnp.zeros((pad,), jnp.float32)])
    ys = jnp.concatenate([y, jnp.full((pad,), 4.0e6, jnp.float32)])
    feats = _pad_tc(sampling_depth_features)
    out = _splat_kernel(xs, ys, feats)
    return out[:, :D].reshape(H, W, D)
